# baseline (speedup 1.0000x reference)
import functools

import jax
import jax.numpy as jnp
from jax import lax
from jax.experimental import pallas as pl
from jax.experimental.pallas import tpu as pltpu
from jax.experimental.pallas import tpu_sc as plsc

N = 10000
E = 320000
D_IN = 128
H = 64

_NC, _NS = 2, 16
_NW = _NC * _NS
_CB = 128
_K = 80
_EPAD = _NW * _K * _CB
_NACC = 10240
_ZROWS = 64
_RPT = _NACC // _NS

_mesh = plsc.VectorSubcoreMesh(core_axis_name="c", subcore_axis_name="s")


@functools.partial(
    pl.kernel,
    out_type=jax.ShapeDtypeStruct((_NC, _NACC), jnp.float32),
    mesh=_mesh,
    scratch_types=[
        pltpu.VMEM((_K, _CB), jnp.int32),
        pltpu.VMEM((_CB,), jnp.float32),
        pltpu.VMEM((_RPT,), jnp.float32),
        pltpu.VMEM_SHARED((_NACC,), jnp.float32),
    ],
    compiler_params=pltpu.CompilerParams(use_tc_tiling_on_sc=False),
)
def _deg_call(dst_hbm, deg_hbm, dst_v, ones_v, zb_v, dacc):
    cid = lax.axis_index("c")
    sid = lax.axis_index("s")
    wid = cid * _NS + sid

    @pl.loop(0, _RPT // 16)
    def _(i):
        zb_v[pl.ds(i * 16, 16)] = jnp.zeros((16,), jnp.float32)

    @pl.loop(0, _CB // 16)
    def _(i):
        ones_v[pl.ds(i * 16, 16)] = jnp.ones((16,), jnp.float32)

    pltpu.sync_copy(zb_v, dacc.at[pl.ds(sid * _RPT, _RPT)])
    pltpu.sync_copy(dst_hbm.at[pl.ds(wid * _K, _K)], dst_v)
    plsc.subcore_barrier()

    @pl.loop(0, _K)
    def _(k):
        pltpu.sync_copy(ones_v, dacc.at[dst_v.at[k]], add=True)

    plsc.subcore_barrier()
    pltpu.sync_copy(dacc.at[pl.ds(sid * _RPT, _RPT)],
                    deg_hbm.at[cid, pl.ds(sid * _RPT, _RPT)])


@functools.partial(
    pl.kernel,
    out_type=jax.ShapeDtypeStruct((_NC, _NACC, H), jnp.float32),
    mesh=_mesh,
    scratch_types=[
        pltpu.VMEM((_K, _CB), jnp.int32),
        pltpu.VMEM((_K, _CB), jnp.int32),
        [pltpu.VMEM((_CB, H), jnp.float32)] * 4,
        pltpu.VMEM((_ZROWS, H), jnp.float32),
        pltpu.VMEM_SHARED((_NACC, H), jnp.float32),
        [pltpu.SemaphoreType.DMA] * 4,
        [pltpu.SemaphoreType.DMA] * 4,
    ],
    compiler_params=pltpu.CompilerParams(use_tc_tiling_on_sc=False),
)
def _agg_call(hs_hbm, src_hbm, dst_hbm, out_hbm,
              src_v, dst_v, rb, zb_v, acc, gsem, ssem):
    cid = lax.axis_index("c")
    sid = lax.axis_index("s")
    wid = cid * _NS + sid
    nb = 4

    @pl.loop(0, _ZROWS)
    def _(r):
        for j in range(H // 16):
            zb_v[r, pl.ds(j * 16, 16)] = jnp.zeros((16,), jnp.float32)

    for t in range(_RPT // _ZROWS):
        pltpu.sync_copy(zb_v, acc.at[pl.ds(sid * _RPT + t * _ZROWS, _ZROWS)])

    pltpu.sync_copy(src_hbm.at[pl.ds(wid * _K, _K)], src_v)
    pltpu.sync_copy(dst_hbm.at[pl.ds(wid * _K, _K)], dst_v)
    plsc.subcore_barrier()

    for b in range(nb):
        pltpu.async_copy(hs_hbm.at[src_v.at[b]], rb[b], gsem[b])

    @pl.loop(0, _K, step=nb)
    def _(k):
        for b in range(nb):
            pltpu.make_async_copy(hs_hbm.at[src_v.at[k + b]], rb[b],
                                  gsem[b]).wait()
            pltpu.async_copy(rb[b], acc.at[dst_v.at[k + b]], ssem[b],
                             add=True)
        for b in range(nb):
            pltpu.make_async_copy(rb[b], acc.at[dst_v.at[k + b]],
                                  ssem[b]).wait()

            @pl.when(k + nb + b < _K)
            def _():
                pltpu.async_copy(hs_hbm.at[src_v.at[k + nb + b]], rb[b],
                                 gsem[b])

    plsc.subcore_barrier()
    pltpu.sync_copy(acc.at[pl.ds(sid * _RPT, _RPT)],
                    out_hbm.at[cid, pl.ds(sid * _RPT, _RPT)])


_RB = 2000


def _mm_first_body(d0_ref, d1_ref, x_ref, w_ref, hs_ref, dinv_ref):
    di = lax.rsqrt(d0_ref[...] + d1_ref[...] + 1.0)
    dinv_ref[...] = di
    hs_ref[...] = di * jnp.dot(x_ref[...], w_ref[...],
                               preferred_element_type=jnp.float32)


_mm_first = pl.pallas_call(
    _mm_first_body,
    grid=(N // _RB,),
    in_specs=[
        pl.BlockSpec((_RB, 1), lambda i: (i, 0)),
        pl.BlockSpec((_RB, 1), lambda i: (i, 0)),
        pl.BlockSpec((_RB, D_IN), lambda i: (i, 0)),
        pl.BlockSpec((D_IN, H), lambda i: (0, 0)),
    ],
    out_specs=[
        pl.BlockSpec((_RB, H), lambda i: (i, 0)),
        pl.BlockSpec((_RB, 1), lambda i: (i, 0)),
    ],
    out_shape=[
        jax.ShapeDtypeStruct((N, H), jnp.float32),
        jax.ShapeDtypeStruct((N, 1), jnp.float32),
    ],
)


def _mm_mid_body(p0_ref, p1_ref, hsp_ref, dinv_ref, b_ref, w_ref, hs_ref):
    di = dinv_ref[...]
    t = di * (p0_ref[...] + p1_ref[...] + hsp_ref[...]) + b_ref[...]
    a = jnp.maximum(t, 0.0)
    hs_ref[...] = di * jnp.dot(a, w_ref[...],
                               preferred_element_type=jnp.float32)


_mm_mid = pl.pallas_call(
    _mm_mid_body,
    grid=(N // _RB,),
    in_specs=[
        pl.BlockSpec((_RB, H), lambda i: (i, 0)),
        pl.BlockSpec((_RB, H), lambda i: (i, 0)),
        pl.BlockSpec((_RB, H), lambda i: (i, 0)),
        pl.BlockSpec((_RB, 1), lambda i: (i, 0)),
        pl.BlockSpec((1, H), lambda i: (0, 0)),
        pl.BlockSpec((H, H), lambda i: (0, 0)),
    ],
    out_specs=pl.BlockSpec((_RB, H), lambda i: (i, 0)),
    out_shape=jax.ShapeDtypeStruct((N, H), jnp.float32),
)


def _fin_body(p0_ref, p1_ref, hs_ref, dinv_ref, b_ref, out_ref):
    out_ref[...] = (dinv_ref[...] * (p0_ref[...] + p1_ref[...] + hs_ref[...])
                    + b_ref[...])


_fin = pl.pallas_call(
    _fin_body,
    grid=(N // _RB,),
    in_specs=[
        pl.BlockSpec((_RB, H), lambda i: (i, 0)),
        pl.BlockSpec((_RB, H), lambda i: (i, 0)),
        pl.BlockSpec((_RB, H), lambda i: (i, 0)),
        pl.BlockSpec((_RB, 1), lambda i: (i, 0)),
        pl.BlockSpec((1, H), lambda i: (0, 0)),
    ],
    out_specs=pl.BlockSpec((_RB, H), lambda i: (i, 0)),
    out_shape=jax.ShapeDtypeStruct((N, H), jnp.float32),
)


def kernel(x, edge_index, W1, b1, W2, b2, W3, b3, W4, b4):
    src = edge_index[0].astype(jnp.int32)
    dst = edge_index[1].astype(jnp.int32)
    pad = _EPAD - E
    srcp = jnp.concatenate([src, jnp.zeros((pad,), jnp.int32)]).reshape(_NW * _K, _CB)
    dstp = jnp.concatenate([dst, jnp.full((pad,), N, jnp.int32)]).reshape(_NW * _K, _CB)

    deg = _deg_call(dstp)
    d0 = deg[0, :N].reshape(N, 1)
    d1 = deg[1, :N].reshape(N, 1)

    hs1, dinv = _mm_first(d0, d1, x, W1)
    p = _agg_call(hs1, srcp, dstp)
    hs2 = _mm_mid(p[0, :N], p[1, :N], hs1, dinv, b1.reshape(1, H), W2)
    p = _agg_call(hs2, srcp, dstp)
    hs3 = _mm_mid(p[0, :N], p[1, :N], hs2, dinv, b2.reshape(1, H), W3)
    p = _agg_call(hs3, srcp, dstp)
    hs4 = _mm_mid(p[0, :N], p[1, :N], hs3, dinv, b3.reshape(1, H), W4)
    p = _agg_call(hs4, srcp, dstp)
    return _fin(p[0, :N], p[1, :N], hs4, dinv, b4.reshape(1, H))


# speedup vs baseline: 1.5349x; 1.5349x over previous
import functools

import jax
import jax.numpy as jnp
import numpy as np
from jax import lax
from jax.experimental import pallas as pl
from jax.experimental.pallas import tpu as pltpu
from jax.experimental.pallas import tpu_sc as plsc

N = 10000
E = 320000
D_IN = 128
H = 64

_NC, _NS = 2, 16
_NW = _NC * _NS
_CB = 128
_K = 80
_EPAD = _NW * _K * _CB
_NACC = 10240
_ZROWS = 128
_RPT = _NACC // _NS
_NB = 4

_Q = np.empty((H,), np.int64)
for _g in range(H // 32):
    for _m in range(16):
        _Q[32 * _g + _m] = 32 * _g + 2 * _m
        _Q[32 * _g + 16 + _m] = 32 * _g + 2 * _m + 1
_PERM = np.zeros((H, H), np.float32)
for _j in range(H):
    _PERM[_j, _Q[_j]] = 1.0

_mesh = plsc.VectorSubcoreMesh(core_axis_name="c", subcore_axis_name="s")


@functools.partial(
    pl.kernel,
    out_type=jax.ShapeDtypeStruct((_NC, _NACC), jnp.float32),
    mesh=_mesh,
    scratch_types=[
        pltpu.VMEM((_K, _CB), jnp.int32),
        pltpu.VMEM((_CB,), jnp.float32),
        pltpu.VMEM((_RPT,), jnp.float32),
        pltpu.VMEM_SHARED((_NACC,), jnp.float32),
    ],
    compiler_params=pltpu.CompilerParams(use_tc_tiling_on_sc=False),
)
def _deg_call(ed_hbm, deg_hbm, dst_v, ones_v, zb_v, dacc):
    cid = lax.axis_index("c")
    sid = lax.axis_index("s")
    wid = cid * _NS + sid

    @pl.loop(0, _RPT // 16)
    def _(i):
        zb_v[pl.ds(i * 16, 16)] = jnp.zeros((16,), jnp.float32)

    @pl.loop(0, _CB // 16)
    def _(i):
        ones_v[pl.ds(i * 16, 16)] = jnp.ones((16,), jnp.float32)

    pltpu.sync_copy(zb_v, dacc.at[pl.ds(sid * _RPT, _RPT)])
    pltpu.sync_copy(ed_hbm.at[pl.ds(wid * _K, _K)], dst_v)

    @pl.loop(0, _K)
    def _(r):
        for j in range(_CB // 16):
            w = dst_v[r, pl.ds(j * 16, 16)]
            dst_v[r, pl.ds(j * 16, 16)] = w & 16383

    plsc.subcore_barrier()

    @pl.loop(0, _K)
    def _(k):
        pltpu.sync_copy(ones_v, dacc.at[dst_v.at[k]], add=True)

    plsc.subcore_barrier()
    pltpu.sync_copy(dacc.at[pl.ds(sid * _RPT, _RPT)],
                    deg_hbm.at[cid, pl.ds(sid * _RPT, _RPT)])


@functools.partial(
    pl.kernel,
    out_type=jax.ShapeDtypeStruct((_NC, _NACC, H), jnp.float32),
    mesh=_mesh,
    scratch_types=[
        pltpu.VMEM((_K * _CB,), jnp.int32),
        pltpu.VMEM((_K, _CB), jnp.int32),
        [pltpu.VMEM((_CB, H), jnp.bfloat16)] * _NB,
        [pltpu.VMEM((_CB, H), jnp.float32)] * _NB,
        pltpu.VMEM_SHARED((_NACC, H), jnp.float32),
        pltpu.VMEM_SHARED((N, H), jnp.bfloat16),
        pltpu.SemaphoreType.DMA,
        pltpu.SemaphoreType.DMA,
    ],
    compiler_params=pltpu.CompilerParams(use_tc_tiling_on_sc=False,
                                         needs_layout_passes=False),
)
def _agg_call(hsb_hbm, ed_hbm, z_hbm, out_hbm,
              src_v, dst_v, rb, cb, acc, hs_s, gsem, ssem):
    cid = lax.axis_index("c")
    sid = lax.axis_index("s")
    wid = cid * _NS + sid
    mhi = jnp.int32(-65536)

    @pl.loop(0, _RPT // _ZROWS)
    def _(t):
        pltpu.sync_copy(z_hbm, acc.at[pl.ds(sid * _RPT + t * _ZROWS, _ZROWS)])

    pltpu.sync_copy(hsb_hbm.at[pl.ds(sid * (N // _NS), N // _NS)],
                    hs_s.at[pl.ds(sid * (N // _NS), N // _NS)])
    pltpu.sync_copy(ed_hbm.at[pl.ds(wid * _K * _CB, _K * _CB)], src_v)

    @pl.loop(0, _K)
    def _(r):
        drow = dst_v.at[r]
        for j in range(_CB // 16):
            w = src_v[pl.ds(r * _CB + j * 16, 16)]
            drow[pl.ds(j * 16, 16)] = w & 16383
            src_v[pl.ds(r * _CB + j * 16, 16)] = lax.shift_right_logical(w, 14)

    plsc.subcore_barrier()

    for b in range(_NB):
        pltpu.async_copy(hs_s.at[src_v.at[pl.ds(b * _CB, _CB)]], rb[b], gsem)

    @pl.loop(0, _K, step=_NB)
    def _(k):
        for b in range(_NB):
            pltpu.make_async_copy(
                hs_s.at[src_v.at[pl.ds((k + b) * _CB, _CB)]], rb[b],
                gsem).wait()
        for b in range(_NB):
            @pl.loop(0, _CB, unroll=8)
            def _(r):
                row = rb[b].at[r]
                crow = cb[b].at[r]
                for g in range(H // 32):
                    w = plsc.bitcast(row[pl.ds(32 * g, 32)], jnp.int32)
                    crow[pl.ds(32 * g, 16)] = plsc.bitcast(
                        w << 16, jnp.float32)
                    crow[pl.ds(32 * g + 16, 16)] = plsc.bitcast(
                        w & mhi, jnp.float32)

            pltpu.async_copy(cb[b], acc.at[dst_v.at[k + b]], ssem, add=True)
        for b in range(_NB):
            pltpu.make_async_copy(cb[b], acc.at[dst_v.at[k + b]],
                                  ssem).wait()
        for b in range(_NB):
            @pl.when(k + _NB + b < _K)
            def _():
                pltpu.async_copy(
                    hs_s.at[src_v.at[pl.ds((k + _NB + b) * _CB, _CB)]],
                    rb[b], gsem)

    plsc.subcore_barrier()
    pltpu.sync_copy(acc.at[pl.ds(sid * _RPT, _RPT)],
                    out_hbm.at[cid, pl.ds(sid * _RPT, _RPT)])


_RB = 2000


def _mm_first_body(d0_ref, d1_ref, x_ref, w_ref, p_ref,
                   hs_ref, hsb_ref, dinv_ref):
    di = lax.rsqrt(d0_ref[...] + d1_ref[...] + 1.0)
    dinv_ref[...] = di
    h = di * jnp.dot(x_ref[...], w_ref[...],
                     preferred_element_type=jnp.float32)
    hs_ref[...] = h
    hsb_ref[...] = jnp.dot(h, p_ref[...],
                           preferred_element_type=jnp.float32
                           ).astype(jnp.bfloat16)


_mm_first = pl.pallas_call(
    _mm_first_body,
    grid=(N // _RB,),
    in_specs=[
        pl.BlockSpec((_RB, 1), lambda i: (i, 0)),
        pl.BlockSpec((_RB, 1), lambda i: (i, 0)),
        pl.BlockSpec((_RB, D_IN), lambda i: (i, 0)),
        pl.BlockSpec((D_IN, H), lambda i: (0, 0)),
        pl.BlockSpec((H, H), lambda i: (0, 0)),
    ],
    out_specs=[
        pl.BlockSpec((_RB, H), lambda i: (i, 0)),
        pl.BlockSpec((_RB, H), lambda i: (i, 0)),
        pl.BlockSpec((_RB, 1), lambda i: (i, 0)),
    ],
    out_shape=[
        jax.ShapeDtypeStruct((N, H), jnp.float32),
        jax.ShapeDtypeStruct((N, H), jnp.bfloat16),
        jax.ShapeDtypeStruct((N, 1), jnp.float32),
    ],
)


def _mm_mid_body(p0_ref, p1_ref, hsp_ref, dinv_ref, b_ref, w_ref, p_ref,
                 hs_ref, hsb_ref):
    di = dinv_ref[...]
    t = di * (p0_ref[...] + p1_ref[...] + hsp_ref[...]) + b_ref[...]
    a = jnp.maximum(t, 0.0)
    h = di * jnp.dot(a, w_ref[...], preferred_element_type=jnp.float32)
    hs_ref[...] = h
    hsb_ref[...] = jnp.dot(h, p_ref[...],
                           preferred_element_type=jnp.float32
                           ).astype(jnp.bfloat16)


_mm_mid = pl.pallas_call(
    _mm_mid_body,
    grid=(N // _RB,),
    in_specs=[
        pl.BlockSpec((_RB, H), lambda i: (i, 0)),
        pl.BlockSpec((_RB, H), lambda i: (i, 0)),
        pl.BlockSpec((_RB, H), lambda i: (i, 0)),
        pl.BlockSpec((_RB, 1), lambda i: (i, 0)),
        pl.BlockSpec((1, H), lambda i: (0, 0)),
        pl.BlockSpec((H, H), lambda i: (0, 0)),
        pl.BlockSpec((H, H), lambda i: (0, 0)),
    ],
    out_specs=[
        pl.BlockSpec((_RB, H), lambda i: (i, 0)),
        pl.BlockSpec((_RB, H), lambda i: (i, 0)),
    ],
    out_shape=[
        jax.ShapeDtypeStruct((N, H), jnp.float32),
        jax.ShapeDtypeStruct((N, H), jnp.bfloat16),
    ],
)


def _fin_body(p0_ref, p1_ref, hs_ref, dinv_ref, b_ref, out_ref):
    out_ref[...] = (dinv_ref[...] * (p0_ref[...] + p1_ref[...] + hs_ref[...])
                    + b_ref[...])


_fin = pl.pallas_call(
    _fin_body,
    grid=(N // _RB,),
    in_specs=[
        pl.BlockSpec((_RB, H), lambda i: (i, 0)),
        pl.BlockSpec((_RB, H), lambda i: (i, 0)),
        pl.BlockSpec((_RB, H), lambda i: (i, 0)),
        pl.BlockSpec((_RB, 1), lambda i: (i, 0)),
        pl.BlockSpec((1, H), lambda i: (0, 0)),
    ],
    out_specs=pl.BlockSpec((_RB, H), lambda i: (i, 0)),
    out_shape=jax.ShapeDtypeStruct((N, H), jnp.float32),
)


def kernel(x, edge_index, W1, b1, W2, b2, W3, b3, W4, b4):
    src = edge_index[0].astype(jnp.int32)
    dst = edge_index[1].astype(jnp.int32)
    pad = _EPAD - E
    ed = src * 16384 + dst
    edflat = jnp.concatenate([ed, jnp.full((pad,), N, jnp.int32)])
    edp = edflat.reshape(_NW * _K, _CB)
    perm = jnp.asarray(_PERM)
    z = jnp.zeros((_ZROWS, H), jnp.float32)

    deg = _deg_call(edp)
    d0 = deg[0, :N].reshape(N, 1)
    d1 = deg[1, :N].reshape(N, 1)

    hs1, hsb, dinv = _mm_first(d0, d1, x, W1, perm)
    p = _agg_call(hsb, edflat, z)
    hs2, hsb = _mm_mid(p[0, :N], p[1, :N], hs1, dinv, b1.reshape(1, H), W2, perm)
    p = _agg_call(hsb, edflat, z)
    hs3, hsb = _mm_mid(p[0, :N], p[1, :N], hs2, dinv, b2.reshape(1, H), W3, perm)
    p = _agg_call(hsb, edflat, z)
    hs4, hsb = _mm_mid(p[0, :N], p[1, :N], hs3, dinv, b3.reshape(1, H), W4, perm)
    p = _agg_call(hsb, edflat, z)
    return _fin(p[0, :N], p[1, :N], hs4, dinv, b4.reshape(1, H))


# speedup vs baseline: 1.6661x; 1.0855x over previous
import functools

import jax
import jax.numpy as jnp
import numpy as np
from jax import lax
from jax.experimental import pallas as pl
from jax.experimental.pallas import tpu as pltpu
from jax.experimental.pallas import tpu_sc as plsc

N = 10000
E = 320000
D_IN = 128
H = 64

_NC, _NS = 2, 16
_NW = _NC * _NS
_CB = 128
_K = 80
_EPAD = _NW * _K * _CB
_NACC = 10240
_ZROWS = 128
_RPT = _NACC // _NS
_NB = 4

_Q = np.empty((H,), np.int64)
for _g in range(H // 32):
    for _m in range(16):
        _Q[32 * _g + _m] = 32 * _g + 2 * _m
        _Q[32 * _g + 16 + _m] = 32 * _g + 2 * _m + 1
_PERM = np.zeros((H, H), np.float32)
for _j in range(H):
    _PERM[_j, _Q[_j]] = 1.0

_mesh = plsc.VectorSubcoreMesh(core_axis_name="c", subcore_axis_name="s")


@functools.partial(
    pl.kernel,
    out_type=jax.ShapeDtypeStruct((_NC, _NACC), jnp.float32),
    mesh=_mesh,
    scratch_types=[
        pltpu.VMEM((_K, _CB), jnp.int32),
        pltpu.VMEM((_CB,), jnp.float32),
        pltpu.VMEM((_RPT,), jnp.float32),
        pltpu.VMEM_SHARED((_NACC,), jnp.float32),
    ],
    compiler_params=pltpu.CompilerParams(use_tc_tiling_on_sc=False),
)
def _deg_call(ed_hbm, deg_hbm, dst_v, ones_v, zb_v, dacc):
    cid = lax.axis_index("c")
    sid = lax.axis_index("s")
    wid = cid * _NS + sid

    @pl.loop(0, _RPT // 16)
    def _(i):
        zb_v[pl.ds(i * 16, 16)] = jnp.zeros((16,), jnp.float32)

    @pl.loop(0, _CB // 16)
    def _(i):
        ones_v[pl.ds(i * 16, 16)] = jnp.ones((16,), jnp.float32)

    pltpu.sync_copy(zb_v, dacc.at[pl.ds(sid * _RPT, _RPT)])
    pltpu.sync_copy(ed_hbm.at[pl.ds(wid * _K, _K)], dst_v)

    @pl.loop(0, _K)
    def _(r):
        for j in range(_CB // 16):
            w = dst_v[r, pl.ds(j * 16, 16)]
            dst_v[r, pl.ds(j * 16, 16)] = w & 16383

    plsc.subcore_barrier()

    @pl.loop(0, _K)
    def _(k):
        pltpu.sync_copy(ones_v, dacc.at[dst_v.at[k]], add=True)

    plsc.subcore_barrier()
    pltpu.sync_copy(dacc.at[pl.ds(sid * _RPT, _RPT)],
                    deg_hbm.at[cid, pl.ds(sid * _RPT, _RPT)])


@functools.partial(
    pl.kernel,
    out_type=jax.ShapeDtypeStruct((_NC, _NACC, H), jnp.float32),
    mesh=_mesh,
    scratch_types=[
        pltpu.VMEM((_K * _CB,), jnp.int32),
        pltpu.VMEM((_K, _CB), jnp.int32),
        [pltpu.VMEM((_CB, H), jnp.bfloat16)] * _NB,
        [pltpu.VMEM((_CB, H), jnp.float32)] * _NB,
        pltpu.VMEM_SHARED((_NACC, H), jnp.float32),
        pltpu.VMEM_SHARED((N, H), jnp.bfloat16),
        pltpu.SemaphoreType.DMA,
        pltpu.SemaphoreType.DMA,
    ],
    compiler_params=pltpu.CompilerParams(use_tc_tiling_on_sc=False,
                                         needs_layout_passes=False),
)
def _agg_call(hsb_hbm, ed_hbm, z_hbm, out_hbm,
              src_v, dst_v, rb, cb, acc, hs_s, gsem, ssem):
    cid = lax.axis_index("c")
    sid = lax.axis_index("s")
    wid = cid * _NS + sid
    mhi = jnp.int32(-65536)

    @pl.loop(0, _RPT // _ZROWS)
    def _(t):
        pltpu.sync_copy(z_hbm, acc.at[pl.ds(sid * _RPT + t * _ZROWS, _ZROWS)])

    pltpu.sync_copy(hsb_hbm.at[pl.ds(sid * (N // _NS), N // _NS)],
                    hs_s.at[pl.ds(sid * (N // _NS), N // _NS)])
    pltpu.sync_copy(ed_hbm.at[pl.ds(wid * _K * _CB, _K * _CB)], src_v)

    @pl.loop(0, _K)
    def _(r):
        drow = dst_v.at[r]
        for j in range(_CB // 16):
            w = src_v[pl.ds(r * _CB + j * 16, 16)]
            drow[pl.ds(j * 16, 16)] = w & 16383
            src_v[pl.ds(r * _CB + j * 16, 16)] = lax.shift_right_logical(w, 14)

    plsc.subcore_barrier()

    for b in range(_NB):
        pltpu.async_copy(hs_s.at[src_v.at[pl.ds(b * _CB, _CB)]], rb[b], gsem)

    @pl.loop(0, _K, step=_NB)
    def _(k):
        for b in range(_NB):
            pltpu.make_async_copy(
                hs_s.at[src_v.at[pl.ds((k + b) * _CB, _CB)]], rb[b],
                gsem).wait()
        for b in range(_NB):
            @pl.loop(0, _CB, unroll=8)
            def _(r):
                row = rb[b].at[r]
                crow = cb[b].at[r]
                for g in range(H // 32):
                    w = plsc.bitcast(row[pl.ds(32 * g, 32)], jnp.int32)
                    crow[pl.ds(32 * g, 16)] = plsc.bitcast(
                        w << 16, jnp.float32)
                    crow[pl.ds(32 * g + 16, 16)] = plsc.bitcast(
                        w & mhi, jnp.float32)

            @pl.when(k + _NB + b < _K)
            def _():
                pltpu.async_copy(
                    hs_s.at[src_v.at[pl.ds((k + _NB + b) * _CB, _CB)]],
                    rb[b], gsem)

            pltpu.async_copy(cb[b], acc.at[dst_v.at[k + b]], ssem, add=True)
        for b in range(_NB):
            pltpu.make_async_copy(cb[b], acc.at[dst_v.at[k + b]],
                                  ssem).wait()

    plsc.subcore_barrier()
    pltpu.sync_copy(acc.at[pl.ds(sid * _RPT, _RPT)],
                    out_hbm.at[cid, pl.ds(sid * _RPT, _RPT)])


_RB = 2000


def _mm_first_body(d0_ref, d1_ref, x_ref, w_ref, p_ref,
                   hs_ref, hsb_ref, dinv_ref):
    di = lax.rsqrt(d0_ref[...] + d1_ref[...] + 1.0)
    dinv_ref[...] = di
    h = di * jnp.dot(x_ref[...], w_ref[...],
                     preferred_element_type=jnp.float32)
    hs_ref[...] = h
    hsb_ref[...] = jnp.dot(h, p_ref[...],
                           preferred_element_type=jnp.float32
                           ).astype(jnp.bfloat16)


_mm_first = pl.pallas_call(
    _mm_first_body,
    grid=(N // _RB,),
    in_specs=[
        pl.BlockSpec((_RB, 1), lambda i: (i, 0)),
        pl.BlockSpec((_RB, 1), lambda i: (i, 0)),
        pl.BlockSpec((_RB, D_IN), lambda i: (i, 0)),
        pl.BlockSpec((D_IN, H), lambda i: (0, 0)),
        pl.BlockSpec((H, H), lambda i: (0, 0)),
    ],
    out_specs=[
        pl.BlockSpec((_RB, H), lambda i: (i, 0)),
        pl.BlockSpec((_RB, H), lambda i: (i, 0)),
        pl.BlockSpec((_RB, 1), lambda i: (i, 0)),
    ],
    out_shape=[
        jax.ShapeDtypeStruct((N, H), jnp.float32),
        jax.ShapeDtypeStruct((N, H), jnp.bfloat16),
        jax.ShapeDtypeStruct((N, 1), jnp.float32),
    ],
)


def _mm_mid_body(p0_ref, p1_ref, hsp_ref, dinv_ref, b_ref, w_ref, p_ref,
                 hs_ref, hsb_ref):
    di = dinv_ref[...]
    t = di * (p0_ref[...] + p1_ref[...] + hsp_ref[...]) + b_ref[...]
    a = jnp.maximum(t, 0.0)
    h = di * jnp.dot(a, w_ref[...], preferred_element_type=jnp.float32)
    hs_ref[...] = h
    hsb_ref[...] = jnp.dot(h, p_ref[...],
                           preferred_element_type=jnp.float32
                           ).astype(jnp.bfloat16)


_mm_mid = pl.pallas_call(
    _mm_mid_body,
    grid=(N // _RB,),
    in_specs=[
        pl.BlockSpec((_RB, H), lambda i: (i, 0)),
        pl.BlockSpec((_RB, H), lambda i: (i, 0)),
        pl.BlockSpec((_RB, H), lambda i: (i, 0)),
        pl.BlockSpec((_RB, 1), lambda i: (i, 0)),
        pl.BlockSpec((1, H), lambda i: (0, 0)),
        pl.BlockSpec((H, H), lambda i: (0, 0)),
        pl.BlockSpec((H, H), lambda i: (0, 0)),
    ],
    out_specs=[
        pl.BlockSpec((_RB, H), lambda i: (i, 0)),
        pl.BlockSpec((_RB, H), lambda i: (i, 0)),
    ],
    out_shape=[
        jax.ShapeDtypeStruct((N, H), jnp.float32),
        jax.ShapeDtypeStruct((N, H), jnp.bfloat16),
    ],
)


def _fin_body(p0_ref, p1_ref, hs_ref, dinv_ref, b_ref, out_ref):
    out_ref[...] = (dinv_ref[...] * (p0_ref[...] + p1_ref[...] + hs_ref[...])
                    + b_ref[...])


_fin = pl.pallas_call(
    _fin_body,
    grid=(N // _RB,),
    in_specs=[
        pl.BlockSpec((_RB, H), lambda i: (i, 0)),
        pl.BlockSpec((_RB, H), lambda i: (i, 0)),
        pl.BlockSpec((_RB, H), lambda i: (i, 0)),
        pl.BlockSpec((_RB, 1), lambda i: (i, 0)),
        pl.BlockSpec((1, H), lambda i: (0, 0)),
    ],
    out_specs=pl.BlockSpec((_RB, H), lambda i: (i, 0)),
    out_shape=jax.ShapeDtypeStruct((N, H), jnp.float32),
)


def kernel(x, edge_index, W1, b1, W2, b2, W3, b3, W4, b4):
    src = edge_index[0].astype(jnp.int32)
    dst = edge_index[1].astype(jnp.int32)
    pad = _EPAD - E
    ed = src * 16384 + dst
    edflat = jnp.concatenate([ed, jnp.full((pad,), N, jnp.int32)])
    edp = edflat.reshape(_NW * _K, _CB)
    perm = jnp.asarray(_PERM)
    z = jnp.zeros((_ZROWS, H), jnp.float32)

    deg = _deg_call(edp)
    d0 = deg[0, :N].reshape(N, 1)
    d1 = deg[1, :N].reshape(N, 1)

    hs1, hsb, dinv = _mm_first(d0, d1, x, W1, perm)
    p = _agg_call(hsb, edflat, z)
    hs2, hsb = _mm_mid(p[0, :N], p[1, :N], hs1, dinv, b1.reshape(1, H), W2, perm)
    p = _agg_call(hsb, edflat, z)
    hs3, hsb = _mm_mid(p[0, :N], p[1, :N], hs2, dinv, b2.reshape(1, H), W3, perm)
    p = _agg_call(hsb, edflat, z)
    hs4, hsb = _mm_mid(p[0, :N], p[1, :N], hs3, dinv, b3.reshape(1, H), W4, perm)
    p = _agg_call(hsb, edflat, z)
    return _fin(p[0, :N], p[1, :N], hs4, dinv, b4.reshape(1, H))
